# initial kernel scaffold (unmeasured)
import jax
import jax.numpy as jnp
from jax import lax
from jax.experimental import pallas as pl
from jax.experimental.pallas import tpu as pltpu


def kernel(
    x,
):
    def body(*refs):
        pass

    out_shape = jax.ShapeDtypeStruct(..., jnp.float32)
    return pl.pallas_call(body, out_shape=out_shape)(...)


# baseline (device time: 1131329 ns/iter reference)
import jax
import jax.numpy as jnp
from jax import lax
from jax.experimental import pallas as pl
from jax.experimental.pallas import tpu as pltpu

M = 16384
NC = 1024


def kernel(x):
    my_y = lax.axis_index("y")
    local_blk = lax.dynamic_slice_in_dim(x, my_y * NC, NC, axis=1).astype(
        jnp.bfloat16
    )
    send_blk = lax.dynamic_slice_in_dim(x, (1 - my_y) * NC, NC, axis=1).astype(
        jnp.bfloat16
    )

    def body(local_ref, send_ref, out_ref, copy_sem, send_sem, recv_sem):
        my_x = lax.axis_index("x")
        my_y = lax.axis_index("y")

        barrier = pltpu.get_barrier_semaphore()
        pl.semaphore_signal(
            barrier,
            inc=1,
            device_id=(my_x, 1 - my_y),
            device_id_type=pl.DeviceIdType.MESH,
        )
        pl.semaphore_wait(barrier, 1)

        local_copy = pltpu.make_async_copy(
            local_ref, out_ref.at[pl.ds(my_y * M, M), :], copy_sem
        )
        local_copy.start()

        rdma = pltpu.make_async_remote_copy(
            src_ref=send_ref,
            dst_ref=out_ref.at[pl.ds(my_y * M, M), :],
            send_sem=send_sem,
            recv_sem=recv_sem,
            device_id=(my_x, 1 - my_y),
            device_id_type=pl.DeviceIdType.MESH,
        )
        rdma.start()
        rdma.wait()
        local_copy.wait()

    return pl.pallas_call(
        body,
        out_shape=jax.ShapeDtypeStruct((2 * M, NC), jnp.bfloat16),
        in_specs=[
            pl.BlockSpec(memory_space=pl.ANY),
            pl.BlockSpec(memory_space=pl.ANY),
        ],
        out_specs=pl.BlockSpec(memory_space=pl.ANY),
        scratch_shapes=[
            pltpu.SemaphoreType.DMA,
            pltpu.SemaphoreType.DMA,
            pltpu.SemaphoreType.DMA,
        ],
        compiler_params=pltpu.CompilerParams(collective_id=0),
    )(local_blk, send_blk)


# device time: 468968 ns/iter; 2.4124x vs baseline; 2.4124x over previous
import jax
import jax.numpy as jnp
from jax import lax
from jax.experimental import pallas as pl
from jax.experimental.pallas import tpu as pltpu

M = 16384
NC = 1024
H = M // 2
C = 16
CH = H // C


def kernel(x):
    my_x = lax.axis_index("x")
    my_y = lax.axis_index("y")

    local_blk = lax.dynamic_slice_in_dim(x, my_y * NC, NC, axis=1).astype(
        jnp.bfloat16
    )
    out_init = lax.dynamic_update_slice(
        jnp.zeros((2 * M, NC), jnp.bfloat16), local_blk, (my_y * M, 0)
    )
    send_half = lax.dynamic_slice(
        x, (my_x * H, (1 - my_y) * NC), (H, NC)
    ).astype(jnp.bfloat16)

    def body(send_ref, init_ref, out_ref, dir_send, dir_recv, fwd_send, fwd_recv):
        my_x = lax.axis_index("x")
        my_y = lax.axis_index("y")
        del init_ref

        barrier = pltpu.get_barrier_semaphore()
        for dev in ((my_x, 1 - my_y), (1 - my_x, my_y)):
            pl.semaphore_signal(
                barrier, inc=1, device_id=dev,
                device_id_type=pl.DeviceIdType.MESH,
            )
        pl.semaphore_wait(barrier, 2)

        directs = []
        for k in range(C):
            rdma = pltpu.make_async_remote_copy(
                src_ref=send_ref.at[pl.ds(k * CH, CH), :],
                dst_ref=out_ref.at[pl.ds(my_y * M + my_x * H + k * CH, CH), :],
                send_sem=dir_send.at[k],
                recv_sem=dir_recv.at[k],
                device_id=(my_x, 1 - my_y),
                device_id_type=pl.DeviceIdType.MESH,
            )
            rdma.start()
            directs.append(rdma)

        forwards = []
        for k in range(C):
            row = (1 - my_y) * M + my_x * H + k * CH
            directs[k].wait_recv()
            fwd = pltpu.make_async_remote_copy(
                src_ref=out_ref.at[pl.ds(row, CH), :],
                dst_ref=out_ref.at[pl.ds(row, CH), :],
                send_sem=fwd_send.at[k],
                recv_sem=fwd_recv.at[k],
                device_id=(1 - my_x, my_y),
                device_id_type=pl.DeviceIdType.MESH,
            )
            fwd.start()
            forwards.append(fwd)

        for k in range(C):
            forwards[k].wait_recv()
        for k in range(C):
            directs[k].wait_send()
            forwards[k].wait_send()

    return pl.pallas_call(
        body,
        out_shape=jax.ShapeDtypeStruct((2 * M, NC), jnp.bfloat16),
        in_specs=[
            pl.BlockSpec(memory_space=pl.ANY),
            pl.BlockSpec(memory_space=pl.ANY),
        ],
        out_specs=pl.BlockSpec(memory_space=pl.ANY),
        scratch_shapes=[
            pltpu.SemaphoreType.DMA((C,)),
            pltpu.SemaphoreType.DMA((C,)),
            pltpu.SemaphoreType.DMA((C,)),
            pltpu.SemaphoreType.DMA((C,)),
        ],
        input_output_aliases={1: 0},
        compiler_params=pltpu.CompilerParams(collective_id=0),
    )(send_half, out_init)


# device time: 365519 ns/iter; 3.0951x vs baseline; 1.2830x over previous
import jax
import jax.numpy as jnp
from jax import lax
from jax.experimental import pallas as pl
from jax.experimental.pallas import tpu as pltpu

M = 16384
NF = 2048
NC = 1024
H = M // 2

C = 16
CH = H // C
CL = 32
CHL = M // CL
NB = 4


def kernel(x):
    def body(
        x_ref,
        out_ref,
        sload,
        sbuf,
        lload,
        lbuf,
        sload_sems,
        lload_sems,
        lstore_sems,
        dir_send,
        dir_recv,
        fwd_send,
        fwd_recv,
    ):
        my_x = lax.axis_index("x")
        my_y = lax.axis_index("y")
        peer_col = (1 - my_y) * NC

        barrier = pltpu.get_barrier_semaphore()
        for dev in ((my_x, 1 - my_y), (1 - my_x, my_y)):
            pl.semaphore_signal(
                barrier, inc=1, device_id=dev,
                device_id_type=pl.DeviceIdType.MESH,
            )
        pl.semaphore_wait(barrier, 2)

        def sload_dma(k):
            return pltpu.make_async_copy(
                x_ref.at[pl.ds(my_x * H + k * CH, CH), pl.ds(peer_col, NC)],
                sload.at[k % NB],
                sload_sems.at[k % NB],
            )

        def lload_dma(k):
            return pltpu.make_async_copy(
                x_ref.at[pl.ds(k * CHL, CHL), pl.ds(my_y * NC, NC)],
                lload.at[k % NB],
                lload_sems.at[k % NB],
            )

        def lstore_dma(k):
            return pltpu.make_async_copy(
                lbuf.at[k % NB],
                out_ref.at[pl.ds(my_y * M + k * CHL, CHL), :],
                lstore_sems.at[k % NB],
            )

        for k in range(NB):
            sload_dma(k).start()
            lload_dma(k).start()

        directs = []
        for k in range(C):
            sload_dma(k).wait()
            if k >= NB:
                directs[k - NB].wait_send()
            sbuf[k % NB] = sload[k % NB].astype(jnp.bfloat16)
            rdma = pltpu.make_async_remote_copy(
                src_ref=sbuf.at[k % NB],
                dst_ref=out_ref.at[pl.ds(my_y * M + my_x * H + k * CH, CH), :],
                send_sem=dir_send.at[k % NB],
                recv_sem=dir_recv.at[k],
                device_id=(my_x, 1 - my_y),
                device_id_type=pl.DeviceIdType.MESH,
            )
            rdma.start()
            directs.append(rdma)
            if k + NB < C:
                sload_dma(k + NB).start()

            for j in (2 * k, 2 * k + 1):
                lload_dma(j).wait()
                if j >= NB:
                    lstore_dma(j - NB).wait()
                lbuf[j % NB] = lload[j % NB].astype(jnp.bfloat16)
                lstore_dma(j).start()
                if j + NB < CL:
                    lload_dma(j + NB).start()

        forwards = []
        for k in range(C):
            row = (1 - my_y) * M + my_x * H + k * CH
            directs[k].wait_recv()
            fwd = pltpu.make_async_remote_copy(
                src_ref=out_ref.at[pl.ds(row, CH), :],
                dst_ref=out_ref.at[pl.ds(row, CH), :],
                send_sem=fwd_send.at[k],
                recv_sem=fwd_recv.at[k],
                device_id=(1 - my_x, my_y),
                device_id_type=pl.DeviceIdType.MESH,
            )
            fwd.start()
            forwards.append(fwd)

        for k in range(C):
            forwards[k].wait_recv()
        for k in range(C - NB, C):
            directs[k].wait_send()
        for j in range(CL - NB, CL):
            lstore_dma(j).wait()
        for k in range(C):
            forwards[k].wait_send()

    return pl.pallas_call(
        body,
        out_shape=jax.ShapeDtypeStruct((2 * M, NC), jnp.bfloat16),
        in_specs=[pl.BlockSpec(memory_space=pl.ANY)],
        out_specs=pl.BlockSpec(memory_space=pl.ANY),
        scratch_shapes=[
            pltpu.VMEM((NB, CH, NC), jnp.float32),
            pltpu.VMEM((NB, CH, NC), jnp.bfloat16),
            pltpu.VMEM((NB, CHL, NC), jnp.float32),
            pltpu.VMEM((NB, CHL, NC), jnp.bfloat16),
            pltpu.SemaphoreType.DMA((NB,)),
            pltpu.SemaphoreType.DMA((NB,)),
            pltpu.SemaphoreType.DMA((NB,)),
            pltpu.SemaphoreType.DMA((NB,)),
            pltpu.SemaphoreType.DMA((C,)),
            pltpu.SemaphoreType.DMA((C,)),
            pltpu.SemaphoreType.DMA((C,)),
        ],
        compiler_params=pltpu.CompilerParams(collective_id=0),
    )(x)


# device time: 364980 ns/iter; 3.0997x vs baseline; 1.0015x over previous
import jax
import jax.numpy as jnp
from jax import lax
from jax.experimental import pallas as pl
from jax.experimental.pallas import tpu as pltpu

M = 16384
NF = 2048
NC = 1024
H = M // 2

C = 16
CH = H // C
CL = 32
CHL = M // CL
NB = 4


def kernel(x):
    def body(
        x_ref,
        out_ref,
        sload,
        sbuf,
        lload,
        lbuf,
        vrecv,
        sload_sems,
        lload_sems,
        lstore_sems,
        vcopy_sems,
        dir_send,
        dir_recv,
        fwd_send,
        fwd_recv,
    ):
        my_x = lax.axis_index("x")
        my_y = lax.axis_index("y")
        peer_col = (1 - my_y) * NC

        barrier = pltpu.get_barrier_semaphore()
        for dev in ((my_x, 1 - my_y), (1 - my_x, my_y)):
            pl.semaphore_signal(
                barrier, inc=1, device_id=dev,
                device_id_type=pl.DeviceIdType.MESH,
            )
        pl.semaphore_wait(barrier, 2)

        def sload_dma(k):
            return pltpu.make_async_copy(
                x_ref.at[pl.ds(my_x * H + k * CH, CH), pl.ds(peer_col, NC)],
                sload.at[k % NB],
                sload_sems.at[k % NB],
            )

        def lload_dma(k):
            return pltpu.make_async_copy(
                x_ref.at[pl.ds(k * CHL, CHL), pl.ds(my_y * NC, NC)],
                lload.at[k % NB],
                lload_sems.at[k % NB],
            )

        def lstore_dma(k):
            return pltpu.make_async_copy(
                lbuf.at[k % NB],
                out_ref.at[pl.ds(my_y * M + k * CHL, CHL), :],
                lstore_sems.at[k % NB],
            )

        for k in range(NB):
            sload_dma(k).start()
            lload_dma(k).start()

        directs = []
        for k in range(C):
            sload_dma(k).wait()
            if k >= NB:
                directs[k - NB].wait_send()
            sbuf[k % NB] = sload[k % NB].astype(jnp.bfloat16)
            rdma = pltpu.make_async_remote_copy(
                src_ref=sbuf.at[k % NB],
                dst_ref=vrecv.at[k],
                send_sem=dir_send.at[k % NB],
                recv_sem=dir_recv.at[k],
                device_id=(my_x, 1 - my_y),
                device_id_type=pl.DeviceIdType.MESH,
            )
            rdma.start()
            directs.append(rdma)
            if k + NB < C:
                sload_dma(k + NB).start()

            for j in (2 * k, 2 * k + 1):
                lload_dma(j).wait()
                if j >= NB:
                    lstore_dma(j - NB).wait()
                lbuf[j % NB] = lload[j % NB].astype(jnp.bfloat16)
                lstore_dma(j).start()
                if j + NB < CL:
                    lload_dma(j + NB).start()

        forwards = []
        for k in range(C):
            row = (1 - my_y) * M + my_x * H + k * CH
            directs[k].wait_recv()
            pltpu.make_async_copy(
                vrecv.at[k], out_ref.at[pl.ds(row, CH), :], vcopy_sems.at[k]
            ).start()
            fwd = pltpu.make_async_remote_copy(
                src_ref=vrecv.at[k],
                dst_ref=out_ref.at[pl.ds(row, CH), :],
                send_sem=fwd_send.at[k],
                recv_sem=fwd_recv.at[k],
                device_id=(1 - my_x, my_y),
                device_id_type=pl.DeviceIdType.MESH,
            )
            fwd.start()
            forwards.append(fwd)

        for k in range(C):
            forwards[k].wait_recv()
        for k in range(C - NB, C):
            directs[k].wait_send()
        for j in range(CL - NB, CL):
            lstore_dma(j).wait()
        for k in range(C):
            forwards[k].wait_send()
            pltpu.make_async_copy(
                vrecv.at[k],
                out_ref.at[pl.ds((1 - my_y) * M + my_x * H + k * CH, CH), :],
                vcopy_sems.at[k],
            ).wait()

    return pl.pallas_call(
        body,
        out_shape=jax.ShapeDtypeStruct((2 * M, NC), jnp.bfloat16),
        in_specs=[pl.BlockSpec(memory_space=pl.ANY)],
        out_specs=pl.BlockSpec(memory_space=pl.ANY),
        scratch_shapes=[
            pltpu.VMEM((NB, CH, NC), jnp.float32),
            pltpu.VMEM((NB, CH, NC), jnp.bfloat16),
            pltpu.VMEM((NB, CHL, NC), jnp.float32),
            pltpu.VMEM((NB, CHL, NC), jnp.bfloat16),
            pltpu.VMEM((C, CH, NC), jnp.bfloat16),
            pltpu.SemaphoreType.DMA((NB,)),
            pltpu.SemaphoreType.DMA((NB,)),
            pltpu.SemaphoreType.DMA((NB,)),
            pltpu.SemaphoreType.DMA((C,)),
            pltpu.SemaphoreType.DMA((NB,)),
            pltpu.SemaphoreType.DMA((C,)),
            pltpu.SemaphoreType.DMA((C,)),
            pltpu.SemaphoreType.DMA((C,)),
        ],
        compiler_params=pltpu.CompilerParams(
            collective_id=0,
            vmem_limit_bytes=56 * 1024 * 1024,
        ),
    )(x)


# device time: 241959 ns/iter; 4.6757x vs baseline; 1.5084x over previous
import jax
import jax.numpy as jnp
from jax import lax
from jax.experimental import pallas as pl
from jax.experimental.pallas import tpu as pltpu

M = 16384
NF = 2048
NC = 1024
H = M // 2

C = 16
CH = H // C
CL = 32
CHL = M // CL
NB = 4


def kernel(x):
    def body(
        x_ref,
        out_ref,
        sload,
        sbuf,
        lload,
        lbuf,
        vrecv,
        sload_sems,
        lload_sems,
        lstore_sems,
        vcopy_sems,
        dir_send,
        dir_recv,
        fwd_send,
        fwd_recv,
    ):
        my_x = lax.axis_index("x")
        my_y = lax.axis_index("y")
        peer_col = (1 - my_y) * NC

        barrier = pltpu.get_barrier_semaphore()
        for dev in ((my_x, 1 - my_y), (1 - my_x, my_y)):
            pl.semaphore_signal(
                barrier, inc=1, device_id=dev,
                device_id_type=pl.DeviceIdType.MESH,
            )
        pl.semaphore_wait(barrier, 2)

        def sload_dma(k):
            return pltpu.make_async_copy(
                x_ref.at[pl.ds(my_x * H + k * CH, CH), pl.ds(peer_col, NC)],
                sload.at[k % NB],
                sload_sems.at[k % NB],
            )

        def lload_dma(k):
            return pltpu.make_async_copy(
                x_ref.at[pl.ds(k * CHL, CHL), pl.ds(my_y * NC, NC)],
                lload.at[k % NB],
                lload_sems.at[k % NB],
            )

        def lstore_dma(k):
            return pltpu.make_async_copy(
                lbuf.at[k % NB],
                out_ref.at[pl.ds(my_y * M + k * CHL, CHL), :],
                lstore_sems.at[k % NB],
            )

        def vcopy_dma(k):
            return pltpu.make_async_copy(
                vrecv.at[k],
                out_ref.at[pl.ds((1 - my_y) * M + my_x * H + k * CH, CH), :],
                vcopy_sems.at[k],
            )

        for k in range(NB):
            sload_dma(k).start()
            lload_dma(k).start()

        directs = []
        for k in range(C):
            sload_dma(k).wait()
            sbuf[k] = sload[k % NB].astype(jnp.bfloat16)
            rdma = pltpu.make_async_remote_copy(
                src_ref=sbuf.at[k],
                dst_ref=vrecv.at[k],
                send_sem=dir_send.at[k],
                recv_sem=dir_recv.at[k],
                device_id=(my_x, 1 - my_y),
                device_id_type=pl.DeviceIdType.MESH,
            )
            rdma.start()
            directs.append(rdma)
            if k + NB < C:
                sload_dma(k + NB).start()

        forwards = []
        for k in range(C):
            directs[k].wait_recv()
            vcopy_dma(k).start()
            fwd = pltpu.make_async_remote_copy(
                src_ref=vrecv.at[k],
                dst_ref=out_ref.at[
                    pl.ds((1 - my_y) * M + my_x * H + k * CH, CH), :
                ],
                send_sem=fwd_send.at[k],
                recv_sem=fwd_recv.at[k],
                device_id=(1 - my_x, my_y),
                device_id_type=pl.DeviceIdType.MESH,
            )
            fwd.start()
            forwards.append(fwd)

            for j in (2 * k, 2 * k + 1):
                lload_dma(j).wait()
                if j >= NB:
                    lstore_dma(j - NB).wait()
                lbuf[j % NB] = lload[j % NB].astype(jnp.bfloat16)
                lstore_dma(j).start()
                if j + NB < CL:
                    lload_dma(j + NB).start()

        for k in range(C):
            forwards[k].wait_recv()
        for k in range(C):
            directs[k].wait_send()
            forwards[k].wait_send()
            vcopy_dma(k).wait()
        for j in range(CL - NB, CL):
            lstore_dma(j).wait()

    return pl.pallas_call(
        body,
        out_shape=jax.ShapeDtypeStruct((2 * M, NC), jnp.bfloat16),
        in_specs=[pl.BlockSpec(memory_space=pl.ANY)],
        out_specs=pl.BlockSpec(memory_space=pl.ANY),
        scratch_shapes=[
            pltpu.VMEM((NB, CH, NC), jnp.float32),
            pltpu.VMEM((C, CH, NC), jnp.bfloat16),
            pltpu.VMEM((NB, CHL, NC), jnp.float32),
            pltpu.VMEM((NB, CHL, NC), jnp.bfloat16),
            pltpu.VMEM((C, CH, NC), jnp.bfloat16),
            pltpu.SemaphoreType.DMA((NB,)),
            pltpu.SemaphoreType.DMA((NB,)),
            pltpu.SemaphoreType.DMA((NB,)),
            pltpu.SemaphoreType.DMA((C,)),
            pltpu.SemaphoreType.DMA((C,)),
            pltpu.SemaphoreType.DMA((C,)),
            pltpu.SemaphoreType.DMA((C,)),
            pltpu.SemaphoreType.DMA((C,)),
        ],
        compiler_params=pltpu.CompilerParams(
            collective_id=0,
            vmem_limit_bytes=60 * 1024 * 1024,
        ),
    )(x)
